# Initial kernel scaffold; baseline (speedup 1.0000x reference)
import jax
import jax.numpy as jnp
from jax.experimental import pallas as pl


def kernel(x, edge_index, W):
    raise NotImplementedError("write your pallas kernel here")


# speedup vs baseline: 26.2964x; 26.2964x over previous
import functools

import jax
import jax.numpy as jnp
from jax import lax
from jax.experimental import pallas as pl
from jax.experimental.pallas import tpu as pltpu
from jax.experimental.pallas import tpu_sc as plsc

N = 10000
E = 320000
C = 128

N_PAD = 10240
NW = 32
EPW = 10240
KC = 80
E_PAD = NW * EPW
ROWS_PER_TILE = N_PAD // 16
BN = 512
GRID_N = N_PAD // BN

_mesh = plsc.VectorSubcoreMesh(core_axis_name="c", subcore_axis_name="s")


def _worker_id():
    return lax.axis_index("c") * 16 + lax.axis_index("s")


@functools.partial(
    pl.kernel,
    out_type=jax.ShapeDtypeStruct((2, N_PAD), jnp.float32),
    mesh=_mesh,
    scratch_types=[
        pltpu.VMEM_SHARED((N_PAD,), jnp.float32),
        pltpu.VMEM((KC, 128), jnp.int32),
        pltpu.VMEM((128,), jnp.float32),
        pltpu.VMEM((ROWS_PER_TILE,), jnp.float32),
    ],
)
def _deg_kernel(col_hbm, out_hbm, acc, col_v, ones_v, buf_v):
    cid = lax.axis_index("c")
    sid = lax.axis_index("s")
    wid = cid * 16 + sid

    z16 = jnp.zeros((16,), jnp.float32)
    for i in range(ROWS_PER_TILE // 16):
        buf_v[pl.ds(i * 16, 16)] = z16
    o16 = jnp.ones((16,), jnp.float32)
    for i in range(8):
        ones_v[pl.ds(i * 16, 16)] = o16
    pltpu.sync_copy(buf_v, acc.at[pl.ds(sid * ROWS_PER_TILE, ROWS_PER_TILE)])
    plsc.subcore_barrier()

    pltpu.sync_copy(col_hbm.at[pl.ds(wid * KC, KC)], col_v)

    def body(j, carry):
        pltpu.sync_copy(ones_v, acc.at[col_v.at[j]], add=True)
        return carry

    lax.fori_loop(0, KC, body, 0)
    plsc.subcore_barrier()

    pltpu.sync_copy(acc.at[pl.ds(sid * ROWS_PER_TILE, ROWS_PER_TILE)], buf_v)
    pltpu.sync_copy(buf_v, out_hbm.at[cid, pl.ds(sid * ROWS_PER_TILE, ROWS_PER_TILE)])


@functools.partial(
    pl.kernel,
    out_type=jax.ShapeDtypeStruct((2, N_PAD, C), jnp.float32),
    mesh=_mesh,
    scratch_types=[
        pltpu.VMEM_SHARED((N_PAD, C), jnp.float32),
        pltpu.VMEM((KC, 128), jnp.int32),
        pltpu.VMEM((KC, 128), jnp.int32),
        pltpu.VMEM((128, C), jnp.float32),
        pltpu.SemaphoreType.DMA,
    ],
)
def _scatter_kernel(y_hbm, row_hbm, col_hbm, out_hbm, acc, row_v, col_v, gbuf, sem):
    cid = lax.axis_index("c")
    sid = lax.axis_index("s")
    wid = cid * 16 + sid

    z16 = jnp.zeros((16,), jnp.float32)

    def zrow(i, carry):
        for cc in range(8):
            gbuf[i, pl.ds(cc * 16, 16)] = z16
        return carry

    lax.fori_loop(0, 128, zrow, 0)
    for k in range(ROWS_PER_TILE // 128):
        pltpu.sync_copy(gbuf, acc.at[pl.ds(sid * ROWS_PER_TILE + k * 128, 128)])
    plsc.subcore_barrier()

    pltpu.sync_copy(row_hbm.at[pl.ds(wid * KC, KC)], row_v)
    pltpu.sync_copy(col_hbm.at[pl.ds(wid * KC, KC)], col_v)

    def body(j, carry):
        pltpu.async_copy(y_hbm.at[row_v.at[j]], gbuf, sem).wait()
        pltpu.sync_copy(gbuf, acc.at[col_v.at[j]], add=True)
        return carry

    lax.fori_loop(0, KC, body, 0)
    plsc.subcore_barrier()

    for k in range(ROWS_PER_TILE // 128):
        base = sid * ROWS_PER_TILE + k * 128
        pltpu.sync_copy(acc.at[pl.ds(base, 128)], gbuf)
        pltpu.sync_copy(gbuf, out_hbm.at[cid, pl.ds(base, 128)])


def _lin_body(x_ref, w_ref, degp_ref, y_ref):
    deg = jnp.sum(degp_ref[...], axis=1, keepdims=True) + 1.0
    dis = lax.rsqrt(deg)
    y = lax.dot_general(x_ref[...], w_ref[...], (((0,), (1,)), ((), ())),
                        preferred_element_type=jnp.float32)
    y_ref[...] = y * dis


_lin_call = pl.pallas_call(
    _lin_body,
    grid=(GRID_N,),
    in_specs=[
        pl.BlockSpec((C, BN), lambda i: (0, i)),
        pl.BlockSpec((C, C), lambda i: (0, 0)),
        pl.BlockSpec((BN, 8), lambda i: (i, 0)),
    ],
    out_specs=pl.BlockSpec((BN, C), lambda i: (i, 0)),
    out_shape=jax.ShapeDtypeStruct((N_PAD, C), jnp.float32),
)


def _final_body(s_ref, y_ref, degp_ref, x_ref, o_ref):
    deg = jnp.sum(degp_ref[...], axis=1, keepdims=True) + 1.0
    dis = lax.rsqrt(deg)
    s = s_ref[0] + s_ref[1] + y_ref[...]
    t = jnp.tanh(s * dis)
    o_ref[...] = t.T + x_ref[...]


_final_call = pl.pallas_call(
    _final_body,
    grid=(GRID_N,),
    in_specs=[
        pl.BlockSpec((2, BN, C), lambda i: (0, i, 0)),
        pl.BlockSpec((BN, C), lambda i: (i, 0)),
        pl.BlockSpec((BN, 8), lambda i: (i, 0)),
        pl.BlockSpec((C, BN), lambda i: (0, i)),
    ],
    out_specs=pl.BlockSpec((C, BN), lambda i: (0, i)),
    out_shape=jax.ShapeDtypeStruct((C, N_PAD), jnp.float32),
)


def kernel(x, edge_index, W):
    row = edge_index[0]
    col = edge_index[1]
    npad = E_PAD - E
    pad_idx = (N + jnp.arange(npad, dtype=jnp.int32) % (N_PAD - N)).astype(jnp.int32)
    row_p = jnp.concatenate([row, pad_idx]).reshape(NW * KC, 128)
    col_p = jnp.concatenate([col, pad_idx]).reshape(NW * KC, 128)

    degp = _deg_kernel(col_p)
    degp8 = jnp.pad(degp.T, ((0, 0), (0, 6)))

    xp = jnp.pad(x, ((0, 0), (0, N_PAD - N)))
    y = _lin_call(xp, W, degp8)

    sp = _scatter_kernel(y, row_p, col_p)

    out = _final_call(sp, y, degp8, xp)
    return out[:, :N]


# speedup vs baseline: 30.4574x; 1.1582x over previous
import functools

import jax
import jax.numpy as jnp
from jax import lax
from jax.experimental import pallas as pl
from jax.experimental.pallas import tpu as pltpu
from jax.experimental.pallas import tpu_sc as plsc

N = 10000
E = 320000
C = 128

N_PAD = 10240
NW = 32
EPW = 10240
KC = 80
E_PAD = NW * EPW
ROWS_PER_TILE = N_PAD // 16
BN = 512
GRID_N = N_PAD // BN

_mesh = plsc.VectorSubcoreMesh(core_axis_name="c", subcore_axis_name="s")


def _worker_id():
    return lax.axis_index("c") * 16 + lax.axis_index("s")


@functools.partial(
    pl.kernel,
    out_type=jax.ShapeDtypeStruct((2, N_PAD), jnp.float32),
    mesh=_mesh,
    scratch_types=[
        pltpu.VMEM_SHARED((N_PAD,), jnp.float32),
        pltpu.VMEM((KC, 128), jnp.int32),
        pltpu.VMEM((128,), jnp.float32),
        pltpu.VMEM((ROWS_PER_TILE,), jnp.float32),
    ],
)
def _deg_kernel(col_hbm, out_hbm, acc, col_v, ones_v, buf_v):
    cid = lax.axis_index("c")
    sid = lax.axis_index("s")
    wid = cid * 16 + sid

    z16 = jnp.zeros((16,), jnp.float32)
    for i in range(ROWS_PER_TILE // 16):
        buf_v[pl.ds(i * 16, 16)] = z16
    o16 = jnp.ones((16,), jnp.float32)
    for i in range(8):
        ones_v[pl.ds(i * 16, 16)] = o16
    pltpu.sync_copy(buf_v, acc.at[pl.ds(sid * ROWS_PER_TILE, ROWS_PER_TILE)])
    plsc.subcore_barrier()

    pltpu.sync_copy(col_hbm.at[pl.ds(wid * KC, KC)], col_v)

    def body(j, carry):
        pltpu.sync_copy(ones_v, acc.at[col_v.at[j]], add=True)
        return carry

    lax.fori_loop(0, KC, body, 0)
    plsc.subcore_barrier()

    pltpu.sync_copy(acc.at[pl.ds(sid * ROWS_PER_TILE, ROWS_PER_TILE)], buf_v)
    pltpu.sync_copy(buf_v, out_hbm.at[cid, pl.ds(sid * ROWS_PER_TILE, ROWS_PER_TILE)])


NBUF = 2
QKC = KC // 5


@functools.partial(
    pl.kernel,
    out_type=jax.ShapeDtypeStruct((2, N_PAD, C), jnp.float32),
    mesh=_mesh,
    scratch_types=[
        pltpu.VMEM_SHARED((N_PAD, C), jnp.float32),
        pltpu.VMEM((QKC, 128), jnp.int32),
        pltpu.VMEM((QKC, 128), jnp.int32),
        pltpu.VMEM((NBUF, 128, C), jnp.float32),
        pltpu.SemaphoreType.DMA,
        pltpu.SemaphoreType.DMA,
        pltpu.SemaphoreType.DMA,
        pltpu.SemaphoreType.DMA,
    ],
)
def _scatter_kernel(y_hbm, row_hbm, col_hbm, out_hbm, acc, row_v, col_v, gbuf,
                    sg0, sg1, ss0, ss1):
    sem_g = [sg0, sg1]
    sem_s = [ss0, ss1]
    cid = lax.axis_index("c")
    sid = lax.axis_index("s")
    wid = cid * 16 + sid

    z16 = jnp.zeros((16,), jnp.float32)

    def zrow(i, carry):
        for cc in range(8):
            gbuf[0, i, pl.ds(cc * 16, 16)] = z16
        return carry

    lax.fori_loop(0, 128, zrow, 0)
    for k in range(ROWS_PER_TILE // 128):
        pltpu.sync_copy(gbuf.at[0], acc.at[pl.ds(sid * ROWS_PER_TILE + k * 128, 128)])
    plsc.subcore_barrier()

    for h in range(5):
        pltpu.sync_copy(row_hbm.at[pl.ds(wid * KC + h * QKC, QKC)], row_v)
        pltpu.sync_copy(col_hbm.at[pl.ds(wid * KC + h * QKC, QKC)], col_v)

        for b in range(NBUF):
            pltpu.async_copy(y_hbm.at[row_v.at[b]], gbuf.at[b], sem_g[b])

        @pl.loop(0, QKC, step=NBUF)
        def _ring(jj):
            for b in range(NBUF):
                j = jj + b
                pltpu.make_async_copy(y_hbm.at[row_v.at[j]], gbuf.at[b], sem_g[b]).wait()
                pltpu.async_copy(gbuf.at[b], acc.at[col_v.at[j]], sem_s[b], add=True)
                pb = (b - 1) % NBUF
                jn = j + NBUF - 1

                @pl.when(jnp.logical_and(j >= 1, jn < QKC))
                def _refill():
                    pltpu.make_async_copy(gbuf.at[pb], acc.at[col_v.at[j]], sem_s[pb]).wait()
                    pltpu.async_copy(y_hbm.at[row_v.at[jn]], gbuf.at[pb], sem_g[pb])

        for b in range(NBUF):
            pltpu.make_async_copy(gbuf.at[b], acc.at[col_v.at[0]], sem_s[b]).wait()
    plsc.subcore_barrier()

    for k in range(ROWS_PER_TILE // 128):
        base = sid * ROWS_PER_TILE + k * 128
        pltpu.sync_copy(acc.at[pl.ds(base, 128)], gbuf.at[0])
        pltpu.sync_copy(gbuf.at[0], out_hbm.at[cid, pl.ds(base, 128)])


def _lin_body(x_ref, w_ref, degp_ref, y_ref):
    deg = jnp.sum(degp_ref[...], axis=1, keepdims=True) + 1.0
    dis = lax.rsqrt(deg)
    y = lax.dot_general(x_ref[...], w_ref[...], (((0,), (1,)), ((), ())),
                        preferred_element_type=jnp.float32)
    y_ref[...] = y * dis


_lin_call = pl.pallas_call(
    _lin_body,
    grid=(GRID_N,),
    in_specs=[
        pl.BlockSpec((C, BN), lambda i: (0, i)),
        pl.BlockSpec((C, C), lambda i: (0, 0)),
        pl.BlockSpec((BN, 8), lambda i: (i, 0)),
    ],
    out_specs=pl.BlockSpec((BN, C), lambda i: (i, 0)),
    out_shape=jax.ShapeDtypeStruct((N_PAD, C), jnp.float32),
)


def _final_body(s_ref, y_ref, degp_ref, x_ref, o_ref):
    deg = jnp.sum(degp_ref[...], axis=1, keepdims=True) + 1.0
    dis = lax.rsqrt(deg)
    s = s_ref[0] + s_ref[1] + y_ref[...]
    t = jnp.tanh(s * dis)
    o_ref[...] = t.T + x_ref[...]


_final_call = pl.pallas_call(
    _final_body,
    grid=(GRID_N,),
    in_specs=[
        pl.BlockSpec((2, BN, C), lambda i: (0, i, 0)),
        pl.BlockSpec((BN, C), lambda i: (i, 0)),
        pl.BlockSpec((BN, 8), lambda i: (i, 0)),
        pl.BlockSpec((C, BN), lambda i: (0, i)),
    ],
    out_specs=pl.BlockSpec((C, BN), lambda i: (0, i)),
    out_shape=jax.ShapeDtypeStruct((C, N_PAD), jnp.float32),
)


def kernel(x, edge_index, W):
    row = edge_index[0]
    col = edge_index[1]
    npad = E_PAD - E
    pad_idx = (N + jnp.arange(npad, dtype=jnp.int32) % (N_PAD - N)).astype(jnp.int32)
    row_p = jnp.concatenate([row, pad_idx]).reshape(NW * KC, 128)
    col_p = jnp.concatenate([col, pad_idx]).reshape(NW * KC, 128)

    degp = _deg_kernel(col_p)
    degp8 = jnp.pad(degp.T, ((0, 0), (0, 6)))

    xp = jnp.pad(x, ((0, 0), (0, N_PAD - N)))
    y = _lin_call(xp, W, degp8)

    sp = _scatter_kernel(y, row_p, col_p)

    out = _final_call(sp, y, degp8, xp)
    return out[:, :N]


# speedup vs baseline: 31.7941x; 1.0439x over previous
import functools

import jax
import jax.numpy as jnp
from jax import lax
from jax.experimental import pallas as pl
from jax.experimental.pallas import tpu as pltpu
from jax.experimental.pallas import tpu_sc as plsc

N = 10000
E = 320000
C = 128

N_PAD = 10240
NW = 32
EPW = 10240
KC = 80
E_PAD = NW * EPW
ROWS_PER_TILE = N_PAD // 16
BN = 512
GRID_N = N_PAD // BN

_mesh = plsc.VectorSubcoreMesh(core_axis_name="c", subcore_axis_name="s")


def _worker_id():
    return lax.axis_index("c") * 16 + lax.axis_index("s")


@functools.partial(
    pl.kernel,
    out_type=jax.ShapeDtypeStruct((2, N_PAD), jnp.float32),
    mesh=_mesh,
    scratch_types=[
        pltpu.VMEM_SHARED((N_PAD,), jnp.float32),
        pltpu.VMEM((KC, 128), jnp.int32),
        pltpu.VMEM((128,), jnp.float32),
        pltpu.VMEM((ROWS_PER_TILE,), jnp.float32),
        pltpu.SemaphoreType.DMA,
    ],
)
def _deg_kernel(col_hbm, out_hbm, acc, col_v, ones_v, buf_v, sem):
    cid = lax.axis_index("c")
    sid = lax.axis_index("s")
    wid = cid * 16 + sid

    z16 = jnp.zeros((16,), jnp.float32)
    for i in range(ROWS_PER_TILE // 16):
        buf_v[pl.ds(i * 16, 16)] = z16
    o16 = jnp.ones((16,), jnp.float32)
    for i in range(8):
        ones_v[pl.ds(i * 16, 16)] = o16
    pltpu.sync_copy(buf_v, acc.at[pl.ds(sid * ROWS_PER_TILE, ROWS_PER_TILE)])
    plsc.subcore_barrier()

    pltpu.sync_copy(col_hbm.at[pl.ds(wid * KC, KC)], col_v)

    def body(j, carry):
        pltpu.async_copy(ones_v, acc.at[col_v.at[j]], sem, add=True)
        return carry

    lax.fori_loop(0, KC, body, 0)

    def drain(j, carry):
        pltpu.make_async_copy(ones_v, acc.at[col_v.at[0]], sem).wait()
        return carry

    lax.fori_loop(0, KC, drain, 0)
    plsc.subcore_barrier()

    pltpu.sync_copy(acc.at[pl.ds(sid * ROWS_PER_TILE, ROWS_PER_TILE)], buf_v)
    pltpu.sync_copy(buf_v, out_hbm.at[cid, pl.ds(sid * ROWS_PER_TILE, ROWS_PER_TILE)])


NBUF = 2
QKC = KC // 5
NGRP = 5


@functools.partial(
    pl.kernel,
    out_type=jax.ShapeDtypeStruct((2, N_PAD, C), jnp.float32),
    mesh=_mesh,
    scratch_types=[
        pltpu.VMEM_SHARED((N_PAD, C), jnp.float32),
        pltpu.VMEM((2, QKC, 128), jnp.int32),
        pltpu.VMEM((2, QKC, 128), jnp.int32),
        pltpu.VMEM((NBUF, 128, C), jnp.float32),
        pltpu.SemaphoreType.DMA,
        pltpu.SemaphoreType.DMA,
        pltpu.SemaphoreType.DMA,
        pltpu.SemaphoreType.DMA,
        pltpu.SemaphoreType.DMA,
        pltpu.SemaphoreType.DMA,
    ],
)
def _scatter_kernel(y_hbm, row_hbm, col_hbm, out_hbm, acc, row_v, col_v, gbuf,
                    sg0, sg1, ss0, ss1, si0, si1):
    sem_g = [sg0, sg1]
    sem_s = [ss0, ss1]
    sem_i = [si0, si1]
    cid = lax.axis_index("c")
    sid = lax.axis_index("s")
    wid = cid * 16 + sid

    z16 = jnp.zeros((16,), jnp.float32)

    def zrow(i, carry):
        for cc in range(8):
            gbuf[0, i, pl.ds(cc * 16, 16)] = z16
        return carry

    lax.fori_loop(0, 128, zrow, 0)
    for k in range(ROWS_PER_TILE // 128):
        pltpu.sync_copy(gbuf.at[0], acc.at[pl.ds(sid * ROWS_PER_TILE + k * 128, 128)])
    plsc.subcore_barrier()

    pltpu.async_copy(row_hbm.at[pl.ds(wid * KC, QKC)], row_v.at[0], sem_i[0])
    pltpu.async_copy(col_hbm.at[pl.ds(wid * KC, QKC)], col_v.at[0], sem_i[1])
    for h in range(NGRP):
        hb = h % 2
        rv = row_v.at[hb]
        cv = col_v.at[hb]
        if h == 0:
            pltpu.make_async_copy(row_hbm.at[pl.ds(wid * KC, QKC)], rv, sem_i[0]).wait()
            pltpu.make_async_copy(col_hbm.at[pl.ds(wid * KC, QKC)], cv, sem_i[1]).wait()
        if h + 1 < NGRP:
            nxt = pl.ds(wid * KC + (h + 1) * QKC, QKC)
            pltpu.async_copy(row_hbm.at[nxt], row_v.at[1 - hb], sem_i[0])
            pltpu.async_copy(col_hbm.at[nxt], col_v.at[1 - hb], sem_i[1])

        for b in range(NBUF):
            pltpu.async_copy(y_hbm.at[rv.at[b]], gbuf.at[b], sem_g[b])

        @pl.loop(0, QKC, step=NBUF)
        def _ring(jj):
            for b in range(NBUF):
                j = jj + b
                pltpu.make_async_copy(y_hbm.at[rv.at[j]], gbuf.at[b], sem_g[b]).wait()
                pltpu.async_copy(gbuf.at[b], acc.at[cv.at[j]], sem_s[b], add=True)
                pb = (b - 1) % NBUF
                jn = j + NBUF - 1

                @pl.when(jnp.logical_and(j >= 1, jn < QKC))
                def _refill():
                    pltpu.make_async_copy(gbuf.at[pb], acc.at[cv.at[j]], sem_s[pb]).wait()
                    pltpu.async_copy(y_hbm.at[rv.at[jn]], gbuf.at[pb], sem_g[pb])

        for b in range(NBUF):
            pltpu.make_async_copy(gbuf.at[b], acc.at[cv.at[0]], sem_s[b]).wait()
        if h + 1 < NGRP:
            nb = 1 - hb
            nxt = pl.ds(wid * KC + (h + 1) * QKC, QKC)
            pltpu.make_async_copy(row_hbm.at[nxt], row_v.at[nb], sem_i[0]).wait()
            pltpu.make_async_copy(col_hbm.at[nxt], col_v.at[nb], sem_i[1]).wait()
    plsc.subcore_barrier()

    for k in range(ROWS_PER_TILE // 128):
        base = sid * ROWS_PER_TILE + k * 128
        pltpu.sync_copy(acc.at[pl.ds(base, 128)], gbuf.at[0])
        pltpu.sync_copy(gbuf.at[0], out_hbm.at[cid, pl.ds(base, 128)])


def _lin_body(x_ref, w_ref, degp_ref, y_ref):
    deg = jnp.sum(degp_ref[...], axis=1, keepdims=True) + 1.0
    dis = lax.rsqrt(deg)
    y = lax.dot_general(x_ref[...], w_ref[...], (((0,), (1,)), ((), ())),
                        preferred_element_type=jnp.float32)
    y_ref[...] = y * dis


_lin_call = pl.pallas_call(
    _lin_body,
    grid=(GRID_N,),
    in_specs=[
        pl.BlockSpec((C, BN), lambda i: (0, i)),
        pl.BlockSpec((C, C), lambda i: (0, 0)),
        pl.BlockSpec((BN, 8), lambda i: (i, 0)),
    ],
    out_specs=pl.BlockSpec((BN, C), lambda i: (i, 0)),
    out_shape=jax.ShapeDtypeStruct((N_PAD, C), jnp.float32),
)


def _final_body(s_ref, y_ref, degp_ref, x_ref, o_ref):
    deg = jnp.sum(degp_ref[...], axis=1, keepdims=True) + 1.0
    dis = lax.rsqrt(deg)
    s = s_ref[0] + s_ref[1] + y_ref[...]
    t = jnp.tanh(s * dis)
    o_ref[...] = t.T + x_ref[...]


_final_call = pl.pallas_call(
    _final_body,
    grid=(GRID_N,),
    in_specs=[
        pl.BlockSpec((2, BN, C), lambda i: (0, i, 0)),
        pl.BlockSpec((BN, C), lambda i: (i, 0)),
        pl.BlockSpec((BN, 8), lambda i: (i, 0)),
        pl.BlockSpec((C, BN), lambda i: (0, i)),
    ],
    out_specs=pl.BlockSpec((C, BN), lambda i: (0, i)),
    out_shape=jax.ShapeDtypeStruct((C, N_PAD), jnp.float32),
)


def kernel(x, edge_index, W):
    row = edge_index[0]
    col = edge_index[1]
    npad = E_PAD - E
    pad_idx = (N + jnp.arange(npad, dtype=jnp.int32) % (N_PAD - N)).astype(jnp.int32)
    row_p = jnp.concatenate([row, pad_idx]).reshape(NW * KC, 128)
    col_p = jnp.concatenate([col, pad_idx]).reshape(NW * KC, 128)

    degp = _deg_kernel(col_p)
    degp8 = jnp.pad(degp.T, ((0, 0), (0, 6)))

    xp = jnp.pad(x, ((0, 0), (0, N_PAD - N)))
    y = _lin_call(xp, W, degp8)

    sp = _scatter_kernel(y, row_p, col_p)

    out = _final_call(sp, y, degp8, xp)
    return out[:, :N]


# speedup vs baseline: 34.0473x; 1.0709x over previous
import functools

import numpy as np
import jax
import jax.numpy as jnp
from jax import lax
from jax.experimental import pallas as pl
from jax.experimental.pallas import tpu as pltpu
from jax.experimental.pallas import tpu_sc as plsc

N = 10000
E = 320000
C = 128

N_PAD = 10240
NW = 32
EPW = 10240
KC = 80
E_PAD = NW * EPW
ROWS_PER_TILE = N_PAD // 16
BN = 512
GRID_N = N_PAD // BN

_mesh = plsc.VectorSubcoreMesh(core_axis_name="c", subcore_axis_name="s")


def _worker_id():
    return lax.axis_index("c") * 16 + lax.axis_index("s")


@functools.partial(
    pl.kernel,
    out_type=jax.ShapeDtypeStruct((2, N_PAD), jnp.float32),
    mesh=_mesh,
    scratch_types=[
        pltpu.VMEM_SHARED((N_PAD,), jnp.float32),
        pltpu.VMEM((KC, 128), jnp.int32),
        pltpu.VMEM((128,), jnp.float32),
        pltpu.VMEM((ROWS_PER_TILE,), jnp.float32),
        pltpu.SemaphoreType.DMA,
    ],
)
def _deg_kernel(ei_hbm, out_hbm, acc, col_v, ones_v, buf_v, sem):
    col_hbm = ei_hbm.at[1]
    cid = lax.axis_index("c")
    sid = lax.axis_index("s")
    wid = cid * 16 + sid

    z16 = jnp.zeros((16,), jnp.float32)
    for i in range(ROWS_PER_TILE // 16):
        buf_v[pl.ds(i * 16, 16)] = z16
    o16 = jnp.ones((16,), jnp.float32)
    for i in range(8):
        ones_v[pl.ds(i * 16, 16)] = o16
    pltpu.sync_copy(buf_v, acc.at[pl.ds(sid * ROWS_PER_TILE, ROWS_PER_TILE)])
    plsc.subcore_barrier()

    pltpu.sync_copy(col_hbm.at[pl.ds(wid * KC, KC)], col_v)

    def body(j, carry):
        pltpu.async_copy(ones_v, acc.at[col_v.at[j]], sem, add=True)
        return carry

    lax.fori_loop(0, KC, body, 0)

    def drain(j, carry):
        pltpu.make_async_copy(ones_v, acc.at[col_v.at[0]], sem).wait()
        return carry

    lax.fori_loop(0, KC, drain, 0)
    plsc.subcore_barrier()

    pltpu.sync_copy(acc.at[pl.ds(sid * ROWS_PER_TILE, ROWS_PER_TILE)], buf_v)
    pltpu.sync_copy(buf_v, out_hbm.at[cid, pl.ds(sid * ROWS_PER_TILE, ROWS_PER_TILE)])


NBUF = 2
QKC = KC // 5
NGRP = 5


@functools.partial(
    pl.kernel,
    out_type=jax.ShapeDtypeStruct((2, N_PAD, C), jnp.float32),
    mesh=_mesh,
    scratch_types=[
        pltpu.VMEM_SHARED((N_PAD, C), jnp.float32),
        pltpu.VMEM((2, QKC, 128), jnp.int32),
        pltpu.VMEM((2, QKC, 128), jnp.int32),
        pltpu.VMEM((NBUF, 128, C), jnp.float32),
        pltpu.SemaphoreType.DMA,
        pltpu.SemaphoreType.DMA,
        pltpu.SemaphoreType.DMA,
        pltpu.SemaphoreType.DMA,
        pltpu.SemaphoreType.DMA,
        pltpu.SemaphoreType.DMA,
    ],
)
def _scatter_kernel(y_hbm, ei_hbm, out_hbm, acc, row_v, col_v, gbuf,
                    sg0, sg1, ss0, ss1, si0, si1):
    row_hbm = ei_hbm.at[0]
    col_hbm = ei_hbm.at[1]
    sem_g = [sg0, sg1]
    sem_s = [ss0, ss1]
    sem_i = [si0, si1]
    cid = lax.axis_index("c")
    sid = lax.axis_index("s")
    wid = cid * 16 + sid

    z16 = jnp.zeros((16,), jnp.float32)

    def zrow(i, carry):
        for cc in range(8):
            gbuf[0, i, pl.ds(cc * 16, 16)] = z16
        return carry

    lax.fori_loop(0, 128, zrow, 0)
    for k in range(ROWS_PER_TILE // 128):
        pltpu.sync_copy(gbuf.at[0], acc.at[pl.ds(sid * ROWS_PER_TILE + k * 128, 128)])
    plsc.subcore_barrier()

    pltpu.async_copy(row_hbm.at[pl.ds(wid * KC, QKC)], row_v.at[0], sem_i[0])
    pltpu.async_copy(col_hbm.at[pl.ds(wid * KC, QKC)], col_v.at[0], sem_i[1])
    for h in range(NGRP):
        hb = h % 2
        rv = row_v.at[hb]
        cv = col_v.at[hb]
        if h == 0:
            pltpu.make_async_copy(row_hbm.at[pl.ds(wid * KC, QKC)], rv, sem_i[0]).wait()
            pltpu.make_async_copy(col_hbm.at[pl.ds(wid * KC, QKC)], cv, sem_i[1]).wait()
        if h + 1 < NGRP:
            nxt = pl.ds(wid * KC + (h + 1) * QKC, QKC)
            pltpu.async_copy(row_hbm.at[nxt], row_v.at[1 - hb], sem_i[0])
            pltpu.async_copy(col_hbm.at[nxt], col_v.at[1 - hb], sem_i[1])

        for b in range(NBUF):
            pltpu.async_copy(y_hbm.at[rv.at[b]], gbuf.at[b], sem_g[b])

        @pl.loop(0, QKC, step=NBUF)
        def _ring(jj):
            for b in range(NBUF):
                j = jj + b
                pltpu.make_async_copy(y_hbm.at[rv.at[j]], gbuf.at[b], sem_g[b]).wait()
                pltpu.async_copy(gbuf.at[b], acc.at[cv.at[j]], sem_s[b], add=True)
                pb = (b - 1) % NBUF
                jn = j + NBUF - 1

                @pl.when(jnp.logical_and(j >= 1, jn < QKC))
                def _refill():
                    pltpu.make_async_copy(gbuf.at[pb], acc.at[cv.at[j]], sem_s[pb]).wait()
                    pltpu.async_copy(y_hbm.at[rv.at[jn]], gbuf.at[pb], sem_g[pb])

        for b in range(NBUF):
            pltpu.make_async_copy(gbuf.at[b], acc.at[cv.at[0]], sem_s[b]).wait()
        if h + 1 < NGRP:
            nb = 1 - hb
            nxt = pl.ds(wid * KC + (h + 1) * QKC, QKC)
            pltpu.make_async_copy(row_hbm.at[nxt], row_v.at[nb], sem_i[0]).wait()
            pltpu.make_async_copy(col_hbm.at[nxt], col_v.at[nb], sem_i[1]).wait()
    plsc.subcore_barrier()

    for k in range(ROWS_PER_TILE // 128):
        base = sid * ROWS_PER_TILE + k * 128
        pltpu.sync_copy(acc.at[pl.ds(base, 128)], gbuf.at[0])
        pltpu.sync_copy(gbuf.at[0], out_hbm.at[cid, pl.ds(base, 128)])


def _lin_body(x_ref, w_ref, degp_ref, y_ref):
    deg = degp_ref[0:1, :] + degp_ref[1:2, :] + 1.0
    dis = lax.rsqrt(deg)
    xs = x_ref[...] * dis
    y_ref[...] = lax.dot_general(xs, w_ref[...], (((0,), (1,)), ((), ())),
                                 preferred_element_type=jnp.float32)


_lin_call = pl.pallas_call(
    _lin_body,
    grid=(GRID_N,),
    in_specs=[
        pl.BlockSpec((C, BN), lambda i: (0, i)),
        pl.BlockSpec((C, C), lambda i: (0, 0)),
        pl.BlockSpec((2, BN), lambda i: (0, i)),
    ],
    out_specs=pl.BlockSpec((BN, C), lambda i: (i, 0)),
    out_shape=jax.ShapeDtypeStruct((N_PAD, C), jnp.float32),
)


def _final_body(s_ref, y_ref, degp_ref, x_ref, o_ref):
    deg = degp_ref[0:1, :] + degp_ref[1:2, :] + 1.0
    dis = lax.rsqrt(deg)
    s = s_ref[0] + s_ref[1] + y_ref[...]
    o_ref[...] = jnp.tanh(s.T * dis) + x_ref[...]


_final_call = pl.pallas_call(
    _final_body,
    grid=(GRID_N,),
    in_specs=[
        pl.BlockSpec((2, BN, C), lambda i: (0, i, 0)),
        pl.BlockSpec((BN, C), lambda i: (i, 0)),
        pl.BlockSpec((2, BN), lambda i: (0, i)),
        pl.BlockSpec((C, BN), lambda i: (0, i)),
    ],
    out_specs=pl.BlockSpec((C, BN), lambda i: (0, i)),
    out_shape=jax.ShapeDtypeStruct((C, N_PAD), jnp.float32),
)


_PAD_IDX = np.broadcast_to(
    (N + np.arange(E_PAD - E, dtype=np.int32) % (N_PAD - N))[None, :], (2, E_PAD - E))


def kernel(x, edge_index, W):
    ei_p = jnp.concatenate(
        [edge_index, jnp.asarray(_PAD_IDX)], axis=1).reshape(2, NW * KC, 128)

    degp = _deg_kernel(ei_p)

    xp = jnp.pad(x, ((0, 0), (0, N_PAD - N)))
    y = _lin_call(xp, W, degp)

    sp = _scatter_kernel(y, ei_p)

    out = _final_call(sp, y, degp, xp)
    return out[:, :N]


# speedup vs baseline: 35.8827x; 1.0539x over previous
import functools

import numpy as np
import jax
import jax.numpy as jnp
from jax import lax
from jax.experimental import pallas as pl
from jax.experimental.pallas import tpu as pltpu
from jax.experimental.pallas import tpu_sc as plsc

N = 10000
E = 320000
C = 128

N_PAD = 10240
NW = 32
EPW = 10240
KC = 80
E_PAD = NW * EPW
ROWS_PER_TILE = N_PAD // 16
BN = 1024
GRID_N = N_PAD // BN

_mesh = plsc.VectorSubcoreMesh(core_axis_name="c", subcore_axis_name="s")


def _worker_id():
    return lax.axis_index("c") * 16 + lax.axis_index("s")


@functools.partial(
    pl.kernel,
    out_type=jax.ShapeDtypeStruct((2, N_PAD), jnp.float32),
    mesh=_mesh,
    scratch_types=[
        pltpu.VMEM_SHARED((N_PAD,), jnp.float32),
        pltpu.VMEM((KC, 128), jnp.int32),
        pltpu.VMEM((128,), jnp.float32),
        pltpu.VMEM((ROWS_PER_TILE,), jnp.float32),
        pltpu.SemaphoreType.DMA,
    ],
)
def _deg_kernel(ei_hbm, out_hbm, acc, col_v, ones_v, buf_v, sem):
    col_hbm = ei_hbm.at[1]
    cid = lax.axis_index("c")
    sid = lax.axis_index("s")
    wid = cid * 16 + sid

    z16 = jnp.zeros((16,), jnp.float32)
    for i in range(ROWS_PER_TILE // 16):
        buf_v[pl.ds(i * 16, 16)] = z16
    o16 = jnp.ones((16,), jnp.float32)
    for i in range(8):
        ones_v[pl.ds(i * 16, 16)] = o16
    pltpu.sync_copy(buf_v, acc.at[pl.ds(sid * ROWS_PER_TILE, ROWS_PER_TILE)])
    plsc.subcore_barrier()

    pltpu.sync_copy(col_hbm.at[pl.ds(wid * KC, KC)], col_v)

    def body(j, carry):
        pltpu.async_copy(ones_v, acc.at[col_v.at[j]], sem, add=True)
        return carry

    lax.fori_loop(0, KC, body, 0)

    def drain(j, carry):
        pltpu.make_async_copy(ones_v, acc.at[col_v.at[0]], sem).wait()
        return carry

    lax.fori_loop(0, KC, drain, 0)
    plsc.subcore_barrier()

    pltpu.sync_copy(acc.at[pl.ds(sid * ROWS_PER_TILE, ROWS_PER_TILE)], buf_v)
    pltpu.sync_copy(buf_v, out_hbm.at[cid, pl.ds(sid * ROWS_PER_TILE, ROWS_PER_TILE)])


NBUF = 2
QKC = KC // 5
NGRP = 5


@functools.partial(
    pl.kernel,
    out_type=jax.ShapeDtypeStruct((2, N_PAD, C), jnp.float32),
    mesh=_mesh,
    scratch_types=[
        pltpu.VMEM_SHARED((N_PAD, C), jnp.float32),
        pltpu.VMEM((2, QKC, 128), jnp.int32),
        pltpu.VMEM((2, QKC, 128), jnp.int32),
        pltpu.VMEM((NBUF, 128, C), jnp.float32),
        pltpu.SemaphoreType.DMA,
        pltpu.SemaphoreType.DMA,
        pltpu.SemaphoreType.DMA,
        pltpu.SemaphoreType.DMA,
        pltpu.SemaphoreType.DMA,
        pltpu.SemaphoreType.DMA,
    ],
)
def _scatter_kernel(y_hbm, ei_hbm, out_hbm, acc, row_v, col_v, gbuf,
                    sg0, sg1, ss0, ss1, si0, si1):
    row_hbm = ei_hbm.at[0]
    col_hbm = ei_hbm.at[1]
    sem_g = [sg0, sg1]
    sem_s = [ss0, ss1]
    sem_i = [si0, si1]
    cid = lax.axis_index("c")
    sid = lax.axis_index("s")
    wid = cid * 16 + sid

    z16 = jnp.zeros((16,), jnp.float32)

    def zrow(i, carry):
        for cc in range(8):
            gbuf[0, i, pl.ds(cc * 16, 16)] = z16
        return carry

    lax.fori_loop(0, 128, zrow, 0)
    for k in range(ROWS_PER_TILE // 128):
        pltpu.sync_copy(gbuf.at[0], acc.at[pl.ds(sid * ROWS_PER_TILE + k * 128, 128)])
    plsc.subcore_barrier()

    pltpu.async_copy(row_hbm.at[pl.ds(wid * KC, QKC)], row_v.at[0], sem_i[0])
    pltpu.async_copy(col_hbm.at[pl.ds(wid * KC, QKC)], col_v.at[0], sem_i[1])
    for h in range(NGRP):
        hb = h % 2
        rv = row_v.at[hb]
        cv = col_v.at[hb]
        if h == 0:
            pltpu.make_async_copy(row_hbm.at[pl.ds(wid * KC, QKC)], rv, sem_i[0]).wait()
            pltpu.make_async_copy(col_hbm.at[pl.ds(wid * KC, QKC)], cv, sem_i[1]).wait()
        if h + 1 < NGRP:
            nxt = pl.ds(wid * KC + (h + 1) * QKC, QKC)
            pltpu.async_copy(row_hbm.at[nxt], row_v.at[1 - hb], sem_i[0])
            pltpu.async_copy(col_hbm.at[nxt], col_v.at[1 - hb], sem_i[1])

        for b in range(NBUF):
            pltpu.async_copy(y_hbm.at[rv.at[b]], gbuf.at[b], sem_g[b])

        @pl.loop(0, QKC, step=NBUF)
        def _ring(jj):
            for b in range(NBUF):
                j = jj + b
                pltpu.make_async_copy(y_hbm.at[rv.at[j]], gbuf.at[b], sem_g[b]).wait()
                pltpu.async_copy(gbuf.at[b], acc.at[cv.at[j]], sem_s[b], add=True)
                pb = (b - 1) % NBUF
                jn = j + NBUF - 1

                @pl.when(jnp.logical_and(j >= 1, jn < QKC))
                def _refill():
                    pltpu.make_async_copy(gbuf.at[pb], acc.at[cv.at[j]], sem_s[pb]).wait()
                    pltpu.async_copy(y_hbm.at[rv.at[jn]], gbuf.at[pb], sem_g[pb])

        for b in range(NBUF):
            pltpu.make_async_copy(gbuf.at[b], acc.at[cv.at[0]], sem_s[b]).wait()
        if h + 1 < NGRP:
            nb = 1 - hb
            nxt = pl.ds(wid * KC + (h + 1) * QKC, QKC)
            pltpu.make_async_copy(row_hbm.at[nxt], row_v.at[nb], sem_i[0]).wait()
            pltpu.make_async_copy(col_hbm.at[nxt], col_v.at[nb], sem_i[1]).wait()
    plsc.subcore_barrier()

    for k in range(ROWS_PER_TILE // 128):
        base = sid * ROWS_PER_TILE + k * 128
        pltpu.sync_copy(acc.at[pl.ds(base, 128)], gbuf.at[0])
        pltpu.sync_copy(gbuf.at[0], out_hbm.at[cid, pl.ds(base, 128)])


def _lin_body(x_ref, w_ref, degp_ref, y_ref):
    deg = degp_ref[0:1, :] + degp_ref[1:2, :] + 1.0
    dis = lax.rsqrt(deg)
    xs = x_ref[...] * dis
    y_ref[...] = lax.dot_general(xs, w_ref[...], (((0,), (1,)), ((), ())),
                                 preferred_element_type=jnp.float32)


_lin_call = pl.pallas_call(
    _lin_body,
    grid=(GRID_N,),
    in_specs=[
        pl.BlockSpec((C, BN), lambda i: (0, i)),
        pl.BlockSpec((C, C), lambda i: (0, 0)),
        pl.BlockSpec((2, BN), lambda i: (0, i)),
    ],
    out_specs=pl.BlockSpec((BN, C), lambda i: (i, 0)),
    out_shape=jax.ShapeDtypeStruct((N_PAD, C), jnp.float32),
)


def _final_body(s_ref, y_ref, degp_ref, x_ref, o_ref):
    deg = degp_ref[0:1, :] + degp_ref[1:2, :] + 1.0
    dis = lax.rsqrt(deg)
    s = s_ref[0] + s_ref[1] + y_ref[...]
    o_ref[...] = jnp.tanh(s.T * dis) + x_ref[...]


_final_call = pl.pallas_call(
    _final_body,
    grid=(GRID_N,),
    in_specs=[
        pl.BlockSpec((2, BN, C), lambda i: (0, i, 0)),
        pl.BlockSpec((BN, C), lambda i: (i, 0)),
        pl.BlockSpec((2, BN), lambda i: (0, i)),
        pl.BlockSpec((C, BN), lambda i: (0, i)),
    ],
    out_specs=pl.BlockSpec((C, BN), lambda i: (0, i)),
    out_shape=jax.ShapeDtypeStruct((C, N_PAD), jnp.float32),
)


_PAD_IDX = np.broadcast_to(
    (N + np.arange(E_PAD - E, dtype=np.int32) % (N_PAD - N))[None, :], (2, E_PAD - E))


def kernel(x, edge_index, W):
    ei_p = jnp.concatenate(
        [edge_index, jnp.asarray(_PAD_IDX)], axis=1).reshape(2, NW * KC, 128)

    degp = _deg_kernel(ei_p)

    xp = jnp.pad(x, ((0, 0), (0, N_PAD - N)))
    y = _lin_call(xp, W, degp)

    sp = _scatter_kernel(y, ei_p)

    out = _final_call(sp, y, degp, xp)
    return out[:, :N]


# speedup vs baseline: 36.6228x; 1.0206x over previous
import functools

import numpy as np
import jax
import jax.numpy as jnp
from jax import lax
from jax.experimental import pallas as pl
from jax.experimental.pallas import tpu as pltpu
from jax.experimental.pallas import tpu_sc as plsc

N = 10000
E = 320000
C = 128

N_PAD = 10240
NW = 32
EPW = 10240
KC = 80
E_PAD = NW * EPW
ROWS_PER_TILE = N_PAD // 16
BN = 2048
GRID_N = N_PAD // BN

_mesh = plsc.VectorSubcoreMesh(core_axis_name="c", subcore_axis_name="s")


def _worker_id():
    return lax.axis_index("c") * 16 + lax.axis_index("s")


@functools.partial(
    pl.kernel,
    out_type=jax.ShapeDtypeStruct((2, N_PAD), jnp.float32),
    mesh=_mesh,
    scratch_types=[
        pltpu.VMEM_SHARED((N_PAD,), jnp.float32),
        pltpu.VMEM((KC, 128), jnp.int32),
        pltpu.VMEM((128,), jnp.float32),
        pltpu.VMEM((ROWS_PER_TILE,), jnp.float32),
        pltpu.SemaphoreType.DMA,
    ],
)
def _deg_kernel(ei_hbm, out_hbm, acc, col_v, ones_v, buf_v, sem):
    col_hbm = ei_hbm.at[1]
    cid = lax.axis_index("c")
    sid = lax.axis_index("s")
    wid = cid * 16 + sid

    z16 = jnp.zeros((16,), jnp.float32)
    for i in range(ROWS_PER_TILE // 16):
        buf_v[pl.ds(i * 16, 16)] = z16
    o16 = jnp.ones((16,), jnp.float32)
    for i in range(8):
        ones_v[pl.ds(i * 16, 16)] = o16
    pltpu.sync_copy(buf_v, acc.at[pl.ds(sid * ROWS_PER_TILE, ROWS_PER_TILE)])
    plsc.subcore_barrier()

    pltpu.sync_copy(col_hbm.at[pl.ds(wid * KC, KC)], col_v)

    def body(j, carry):
        pltpu.async_copy(ones_v, acc.at[col_v.at[j]], sem, add=True)
        return carry

    lax.fori_loop(0, KC, body, 0)

    def drain(j, carry):
        pltpu.make_async_copy(ones_v, acc.at[col_v.at[0]], sem).wait()
        return carry

    lax.fori_loop(0, KC, drain, 0)
    plsc.subcore_barrier()

    pltpu.sync_copy(acc.at[pl.ds(sid * ROWS_PER_TILE, ROWS_PER_TILE)], buf_v)
    pltpu.sync_copy(buf_v, out_hbm.at[cid, pl.ds(sid * ROWS_PER_TILE, ROWS_PER_TILE)])


NBUF = 2
QKC = KC // 5
NGRP = 5


@functools.partial(
    pl.kernel,
    out_type=jax.ShapeDtypeStruct((2, N_PAD, C), jnp.float32),
    mesh=_mesh,
    scratch_types=[
        pltpu.VMEM_SHARED((N_PAD, C), jnp.float32),
        pltpu.VMEM((2, QKC, 128), jnp.int32),
        pltpu.VMEM((2, QKC, 128), jnp.int32),
        pltpu.VMEM((NBUF, 128, C), jnp.float32),
        pltpu.SemaphoreType.DMA,
        pltpu.SemaphoreType.DMA,
        pltpu.SemaphoreType.DMA,
        pltpu.SemaphoreType.DMA,
        pltpu.SemaphoreType.DMA,
        pltpu.SemaphoreType.DMA,
    ],
)
def _scatter_kernel(y_hbm, ei_hbm, out_hbm, acc, row_v, col_v, gbuf,
                    sg0, sg1, ss0, ss1, si0, si1):
    row_hbm = ei_hbm.at[0]
    col_hbm = ei_hbm.at[1]
    sem_g = [sg0, sg1]
    sem_s = [ss0, ss1]
    sem_i = [si0, si1]
    cid = lax.axis_index("c")
    sid = lax.axis_index("s")
    wid = cid * 16 + sid

    z16 = jnp.zeros((16,), jnp.float32)

    def zrow(i, carry):
        for cc in range(8):
            gbuf[0, i, pl.ds(cc * 16, 16)] = z16
        return carry

    lax.fori_loop(0, 128, zrow, 0)
    for k in range(ROWS_PER_TILE // 128):
        pltpu.sync_copy(gbuf.at[0], acc.at[pl.ds(sid * ROWS_PER_TILE + k * 128, 128)])
    plsc.subcore_barrier()

    pltpu.async_copy(row_hbm.at[pl.ds(wid * KC, QKC)], row_v.at[0], sem_i[0])
    pltpu.async_copy(col_hbm.at[pl.ds(wid * KC, QKC)], col_v.at[0], sem_i[1])
    for h in range(NGRP):
        hb = h % 2
        rv = row_v.at[hb]
        cv = col_v.at[hb]
        if h == 0:
            pltpu.make_async_copy(row_hbm.at[pl.ds(wid * KC, QKC)], rv, sem_i[0]).wait()
            pltpu.make_async_copy(col_hbm.at[pl.ds(wid * KC, QKC)], cv, sem_i[1]).wait()
        if h + 1 < NGRP:
            nxt = pl.ds(wid * KC + (h + 1) * QKC, QKC)
            pltpu.async_copy(row_hbm.at[nxt], row_v.at[1 - hb], sem_i[0])
            pltpu.async_copy(col_hbm.at[nxt], col_v.at[1 - hb], sem_i[1])

        for b in range(NBUF):
            pltpu.async_copy(y_hbm.at[rv.at[b]], gbuf.at[b], sem_g[b])

        @pl.loop(0, QKC, step=NBUF)
        def _ring(jj):
            for b in range(NBUF):
                j = jj + b
                pltpu.make_async_copy(y_hbm.at[rv.at[j]], gbuf.at[b], sem_g[b]).wait()
                pltpu.async_copy(gbuf.at[b], acc.at[cv.at[j]], sem_s[b], add=True)
                pb = (b - 1) % NBUF
                jn = j + NBUF - 1

                @pl.when(jnp.logical_and(j >= 1, jn < QKC))
                def _refill():
                    pltpu.make_async_copy(gbuf.at[pb], acc.at[cv.at[j]], sem_s[pb]).wait()
                    pltpu.async_copy(y_hbm.at[rv.at[jn]], gbuf.at[pb], sem_g[pb])

        for b in range(NBUF):
            pltpu.make_async_copy(gbuf.at[b], acc.at[cv.at[0]], sem_s[b]).wait()
        if h + 1 < NGRP:
            nb = 1 - hb
            nxt = pl.ds(wid * KC + (h + 1) * QKC, QKC)
            pltpu.make_async_copy(row_hbm.at[nxt], row_v.at[nb], sem_i[0]).wait()
            pltpu.make_async_copy(col_hbm.at[nxt], col_v.at[nb], sem_i[1]).wait()
    plsc.subcore_barrier()

    for k in range(ROWS_PER_TILE // 128):
        base = sid * ROWS_PER_TILE + k * 128
        pltpu.sync_copy(acc.at[pl.ds(base, 128)], gbuf.at[0])
        pltpu.sync_copy(gbuf.at[0], out_hbm.at[cid, pl.ds(base, 128)])


def _lin_body(x_ref, w_ref, degp_ref, y_ref):
    deg = degp_ref[0:1, :] + degp_ref[1:2, :] + 1.0
    dis = lax.rsqrt(deg)
    xs = x_ref[...] * dis
    y_ref[...] = lax.dot_general(xs, w_ref[...], (((0,), (1,)), ((), ())),
                                 preferred_element_type=jnp.float32)


_lin_call = pl.pallas_call(
    _lin_body,
    grid=(GRID_N,),
    in_specs=[
        pl.BlockSpec((C, BN), lambda i: (0, i)),
        pl.BlockSpec((C, C), lambda i: (0, 0)),
        pl.BlockSpec((2, BN), lambda i: (0, i)),
    ],
    out_specs=pl.BlockSpec((BN, C), lambda i: (i, 0)),
    out_shape=jax.ShapeDtypeStruct((N_PAD, C), jnp.float32),
)


def _final_body(s_ref, y_ref, degp_ref, x_ref, o_ref):
    deg = degp_ref[0:1, :] + degp_ref[1:2, :] + 1.0
    dis = lax.rsqrt(deg)
    s = s_ref[0] + s_ref[1] + y_ref[...]
    o_ref[...] = jnp.tanh(s.T * dis) + x_ref[...]


_final_call = pl.pallas_call(
    _final_body,
    grid=(GRID_N,),
    in_specs=[
        pl.BlockSpec((2, BN, C), lambda i: (0, i, 0)),
        pl.BlockSpec((BN, C), lambda i: (i, 0)),
        pl.BlockSpec((2, BN), lambda i: (0, i)),
        pl.BlockSpec((C, BN), lambda i: (0, i)),
    ],
    out_specs=pl.BlockSpec((C, BN), lambda i: (0, i)),
    out_shape=jax.ShapeDtypeStruct((C, N_PAD), jnp.float32),
)


_PAD_IDX = np.broadcast_to(
    (N + np.arange(E_PAD - E, dtype=np.int32) % (N_PAD - N))[None, :], (2, E_PAD - E))


def kernel(x, edge_index, W):
    ei_p = jnp.concatenate(
        [edge_index, jnp.asarray(_PAD_IDX)], axis=1).reshape(2, NW * KC, 128)

    degp = _deg_kernel(ei_p)

    xp = jnp.pad(x, ((0, 0), (0, N_PAD - N)))
    y = _lin_call(xp, W, degp)

    sp = _scatter_kernel(y, ei_p)

    out = _final_call(sp, y, degp, xp)
    return out[:, :N]


# speedup vs baseline: 36.7118x; 1.0024x over previous
import functools

import numpy as np
import jax
import jax.numpy as jnp
from jax import lax
from jax.experimental import pallas as pl
from jax.experimental.pallas import tpu as pltpu
from jax.experimental.pallas import tpu_sc as plsc

N = 10000
E = 320000
C = 128

N_PAD = 10240
NW = 32
EPW = 10240
KC = 80
E_PAD = NW * EPW
ROWS_PER_TILE = N_PAD // 16
BN = 2048
GRID_N = N_PAD // BN

_mesh = plsc.VectorSubcoreMesh(core_axis_name="c", subcore_axis_name="s")


@functools.partial(
    pl.kernel,
    out_type=jax.ShapeDtypeStruct((2, N_PAD), jnp.float32),
    mesh=_mesh,
    scratch_types=[
        pltpu.VMEM_SHARED((N_PAD,), jnp.float32),
        pltpu.VMEM((KC, 128), jnp.int32),
        pltpu.VMEM((128,), jnp.float32),
        pltpu.VMEM((ROWS_PER_TILE,), jnp.float32),
        pltpu.SemaphoreType.DMA,
    ],
)
def _deg_kernel(ei_hbm, out_hbm, acc, col_v, ones_v, buf_v, sem):
    col_hbm = ei_hbm.at[1]
    cid = lax.axis_index("c")
    sid = lax.axis_index("s")
    wid = cid * 16 + sid

    z16 = jnp.zeros((16,), jnp.float32)
    for i in range(ROWS_PER_TILE // 16):
        buf_v[pl.ds(i * 16, 16)] = z16
    o16 = jnp.ones((16,), jnp.float32)
    for i in range(8):
        ones_v[pl.ds(i * 16, 16)] = o16
    pltpu.sync_copy(buf_v, acc.at[pl.ds(sid * ROWS_PER_TILE, ROWS_PER_TILE)])
    plsc.subcore_barrier()

    pltpu.sync_copy(col_hbm.at[pl.ds(wid * KC, KC)], col_v)

    def body(j, carry):
        pltpu.async_copy(ones_v, acc.at[col_v.at[j]], sem, add=True)
        return carry

    lax.fori_loop(0, KC, body, 0)

    def drain(j, carry):
        pltpu.make_async_copy(ones_v, acc.at[col_v.at[0]], sem).wait()
        return carry

    lax.fori_loop(0, KC, drain, 0)
    plsc.subcore_barrier()

    pltpu.sync_copy(acc.at[pl.ds(sid * ROWS_PER_TILE, ROWS_PER_TILE)], buf_v)
    pltpu.sync_copy(buf_v, out_hbm.at[cid, pl.ds(sid * ROWS_PER_TILE, ROWS_PER_TILE)])


NBUF = 2
QKC = KC // 5
NGRP = 5


@functools.partial(
    pl.kernel,
    out_type=jax.ShapeDtypeStruct((2, N_PAD, C), jnp.float32),
    mesh=_mesh,
    scratch_types=[
        pltpu.VMEM_SHARED((N_PAD, C), jnp.float32),
        pltpu.VMEM((2, QKC, 128), jnp.int32),
        pltpu.VMEM((2, QKC, 128), jnp.int32),
        pltpu.VMEM((NBUF, 128, C), jnp.float32),
        pltpu.SemaphoreType.DMA,
        pltpu.SemaphoreType.DMA,
        pltpu.SemaphoreType.DMA,
        pltpu.SemaphoreType.DMA,
        pltpu.SemaphoreType.DMA,
        pltpu.SemaphoreType.DMA,
    ],
)
def _scatter_kernel(y_hbm, ei_hbm, out_hbm, acc, row_v, col_v, gbuf,
                    sg0, sg1, ss0, ss1, si0, si1):
    row_hbm = ei_hbm.at[0]
    col_hbm = ei_hbm.at[1]
    sem_g = [sg0, sg1]
    sem_s = [ss0, ss1]
    sem_i = [si0, si1]
    cid = lax.axis_index("c")
    sid = lax.axis_index("s")
    wid = cid * 16 + sid

    z16 = jnp.zeros((16,), jnp.float32)

    def zrow(i, carry):
        for cc in range(8):
            gbuf[0, i, pl.ds(cc * 16, 16)] = z16
        return carry

    lax.fori_loop(0, 128, zrow, 0)
    for k in range(ROWS_PER_TILE // 128):
        pltpu.sync_copy(gbuf.at[0], acc.at[pl.ds(sid * ROWS_PER_TILE + k * 128, 128)])
    plsc.subcore_barrier()

    pltpu.async_copy(row_hbm.at[pl.ds(wid * KC, QKC)], row_v.at[0], sem_i[0])
    pltpu.async_copy(col_hbm.at[pl.ds(wid * KC, QKC)], col_v.at[0], sem_i[1])
    for h in range(NGRP):
        hb = h % 2
        rv = row_v.at[hb]
        cv = col_v.at[hb]
        if h == 0:
            pltpu.make_async_copy(row_hbm.at[pl.ds(wid * KC, QKC)], rv, sem_i[0]).wait()
            pltpu.make_async_copy(col_hbm.at[pl.ds(wid * KC, QKC)], cv, sem_i[1]).wait()
        if h + 1 < NGRP:
            nxt = pl.ds(wid * KC + (h + 1) * QKC, QKC)
            pltpu.async_copy(row_hbm.at[nxt], row_v.at[1 - hb], sem_i[0])
            pltpu.async_copy(col_hbm.at[nxt], col_v.at[1 - hb], sem_i[1])

        for b in range(NBUF):
            pltpu.async_copy(y_hbm.at[rv.at[b]], gbuf.at[b], sem_g[b])

        @pl.loop(0, QKC, step=NBUF)
        def _ring(jj):
            for b in range(NBUF):
                j = jj + b
                pltpu.make_async_copy(y_hbm.at[rv.at[j]], gbuf.at[b], sem_g[b]).wait()
                pltpu.async_copy(gbuf.at[b], acc.at[cv.at[j]], sem_s[b], add=True)
                pb = (b - 1) % NBUF
                jn = j + NBUF - 1

                @pl.when(jnp.logical_and(j >= 1, jn < QKC))
                def _refill():
                    pltpu.make_async_copy(gbuf.at[pb], acc.at[cv.at[j]], sem_s[pb]).wait()
                    pltpu.async_copy(y_hbm.at[rv.at[jn]], gbuf.at[pb], sem_g[pb])

        for b in range(NBUF):
            pltpu.make_async_copy(gbuf.at[b], acc.at[cv.at[0]], sem_s[b]).wait()
        if h + 1 < NGRP:
            nb = 1 - hb
            nxt = pl.ds(wid * KC + (h + 1) * QKC, QKC)
            pltpu.make_async_copy(row_hbm.at[nxt], row_v.at[nb], sem_i[0]).wait()
            pltpu.make_async_copy(col_hbm.at[nxt], col_v.at[nb], sem_i[1]).wait()
    plsc.subcore_barrier()

    for k in range(ROWS_PER_TILE // 128):
        base = sid * ROWS_PER_TILE + k * 128
        pltpu.sync_copy(acc.at[pl.ds(base, 128)], gbuf.at[0])
        pltpu.sync_copy(gbuf.at[0], out_hbm.at[cid, pl.ds(base, 128)])


def _lin_body(x_ref, w_ref, degp_ref, y_ref):
    deg = degp_ref[0:1, :] + degp_ref[1:2, :] + 1.0
    dis = lax.rsqrt(deg)
    xs = x_ref[...] * dis
    y_ref[...] = lax.dot_general(xs, w_ref[...], (((0,), (1,)), ((), ())),
                                 preferred_element_type=jnp.float32)


_lin_call = pl.pallas_call(
    _lin_body,
    grid=(GRID_N,),
    in_specs=[
        pl.BlockSpec((C, BN), lambda i: (0, i)),
        pl.BlockSpec((C, C), lambda i: (0, 0)),
        pl.BlockSpec((2, BN), lambda i: (0, i)),
    ],
    out_specs=pl.BlockSpec((BN, C), lambda i: (i, 0)),
    out_shape=jax.ShapeDtypeStruct((N_PAD, C), jnp.float32),
)


def _final_body(s_ref, y_ref, degp_ref, x_ref, o_ref):
    deg = degp_ref[0:1, :] + degp_ref[1:2, :] + 1.0
    dis = lax.rsqrt(deg)
    s = s_ref[0] + s_ref[1] + y_ref[...]
    o_ref[...] = jnp.tanh(s.T * dis) + x_ref[...]


_final_call = pl.pallas_call(
    _final_body,
    grid=(GRID_N,),
    in_specs=[
        pl.BlockSpec((2, BN, C), lambda i: (0, i, 0)),
        pl.BlockSpec((BN, C), lambda i: (i, 0)),
        pl.BlockSpec((2, BN), lambda i: (0, i)),
        pl.BlockSpec((C, BN), lambda i: (0, i)),
    ],
    out_specs=pl.BlockSpec((C, BN), lambda i: (0, i)),
    out_shape=jax.ShapeDtypeStruct((C, N_PAD), jnp.float32),
)


_PAD_IDX = np.broadcast_to(
    (N + np.arange(E_PAD - E, dtype=np.int32) % (N_PAD - N))[None, :], (2, E_PAD - E))


def kernel(x, edge_index, W):
    ei_p = jnp.concatenate(
        [edge_index, jnp.asarray(_PAD_IDX)], axis=1).reshape(2, NW * KC, 128)

    degp = _deg_kernel(ei_p)

    xp = jnp.pad(x, ((0, 0), (0, N_PAD - N)))
    y = _lin_call(xp, W, degp)

    sp = _scatter_kernel(y, ei_p)

    out = _final_call(sp, y, degp, xp)
    return out[:, :N]
